# baseline (speedup 1.0000x reference)
import functools

import jax
import jax.numpy as jnp
from jax import lax
from jax.experimental import pallas as pl
from jax.experimental.pallas import tpu as pltpu
from jax.experimental.pallas import tpu_sc as plsc

_VOCAB = 49408
_D = 768
_S = 77
_B = 4096

_NC = 2
_NS = 16
_NW = _NC * _NS
_BPW = _B // _NW
_CH = 32
_GPW = _BPW // _CH
_NU = _S * _GPW
_LANES = _D // 16
_NB = 4


def _sc_body(ids_hbm, w_hbm, pos_hbm, out_hbm,
             idx2, prow2, rows0, rows1, rows2, rows3,
             gsem0, gsem1, gsem2, gsem3, ssem0, ssem1, ssem2, ssem3):
    wid = lax.axis_index("s") * _NC + lax.axis_index("c")
    wb = wid * _BPW
    bufs = (rows0, rows1, rows2, rows3)
    gsems = (gsem0, gsem1, gsem2, gsem3)
    ssems = (ssem0, ssem1, ssem2, ssem3)

    def stage_group(q):
        sel = lax.rem(q, 2)
        pltpu.sync_copy(ids_hbm.at[q, pl.ds(wb, _BPW)], idx2.at[sel])
        pltpu.sync_copy(pos_hbm.at[q], prow2.at[sel])

    def gather_start(t, b):
        q = t // _GPW
        g = t % _GPW
        sel = lax.rem(q, 2)
        pltpu.async_copy(
            w_hbm.at[idx2.at[sel, pl.ds(g * _CH, _CH)]], bufs[b], gsems[b])

    stage_group(0)
    for b in range(_NB - 1):
        gather_start(b, b)

    def quad_body(p, _):
        for b in range(_NB):
            t = _NB * p + b

            pltpu.make_async_copy(w_hbm.at[idx2.at[0, pl.ds(0, _CH)]],
                                  bufs[b], gsems[b]).wait()

            tp = t + (_NB - 1)
            pb = (b + _NB - 1) % _NB

            @pl.when(tp < _NU)
            def _():
                if b == 1:
                    stage_group(tp // _GPW)

                @pl.when(tp >= _NB)
                def _():
                    pltpu.make_async_copy(
                        bufs[pb], out_hbm.at[pl.ds(0, _CH), 0], ssems[pb]).wait()
                gather_start(tp, pb)

            q = t // _GPW
            g = t % _GPW
            sel = lax.rem(q, 2)

            for jb in range(_LANES // 8):
                base = jb * 128
                pvecs = tuple(
                    prow2[sel, pl.ds(base + k * 16, 16)] for k in range(8))

                def row_body(i, pv, _b=b, _base=base):
                    for k in range(8):
                        sl = pl.ds(_base + k * 16, 16)
                        bufs[_b][i, sl] = bufs[_b][i, sl] + pv[k]
                    return pv

                lax.fori_loop(0, _CH, row_body, pvecs, unroll=2)

            pltpu.async_copy(
                bufs[b], out_hbm.at[pl.ds(wb + g * _CH, _CH), q], ssems[b])
        return 0

    lax.fori_loop(0, _NU // _NB, quad_body, 0, unroll=False)

    for b in range(_NB):
        pltpu.make_async_copy(
            bufs[b], out_hbm.at[pl.ds(0, _CH), 0], ssems[b]).wait()


@jax.jit
def _embed(input_ids, weight, position_embedding):
    ids = jnp.swapaxes(input_ids.astype(jnp.int32), 0, 1)
    mesh = plsc.VectorSubcoreMesh(
        core_axis_name="c", subcore_axis_name="s",
        num_cores=_NC, num_subcores=_NS,
    )
    run = pl.kernel(
        _sc_body,
        out_type=jax.ShapeDtypeStruct((_B, _S, _D), jnp.float32),
        mesh=mesh,
        scratch_types=[
            pltpu.VMEM((2, _BPW), jnp.int32),
            pltpu.VMEM((2, _D), jnp.float32),
            pltpu.VMEM((_CH, _D), jnp.float32),
            pltpu.VMEM((_CH, _D), jnp.float32),
            pltpu.VMEM((_CH, _D), jnp.float32),
            pltpu.VMEM((_CH, _D), jnp.float32),
            pltpu.SemaphoreType.DMA,
            pltpu.SemaphoreType.DMA,
            pltpu.SemaphoreType.DMA,
            pltpu.SemaphoreType.DMA,
            pltpu.SemaphoreType.DMA,
            pltpu.SemaphoreType.DMA,
            pltpu.SemaphoreType.DMA,
            pltpu.SemaphoreType.DMA,
        ],
    )
    return run(ids, weight, position_embedding)


def kernel(input_ids, weight, position_embedding):
    return _embed(input_ids, weight, position_embedding)
